# Initial kernel scaffold; baseline (speedup 1.0000x reference)
import jax
import jax.numpy as jnp
from jax.experimental import pallas as pl


def kernel(x, edge_index, i, W1, b1, W2, b2, Wo, bo):
    raise NotImplementedError("write your pallas kernel here")


# speedup vs baseline: 3.9026x; 3.9026x over previous
import functools

import jax
import jax.numpy as jnp
from jax import lax
from jax.experimental import pallas as pl
from jax.experimental.pallas import tpu as pltpu
from jax.experimental.pallas import tpu_sc as plsc

NC = 2
NS = 16
NW = NC * NS
CH = 128


def _make_agg(n, n_acc, d, epw):
    n_chunks = epw // CH
    rz = n_acc // NS
    last = n - (NS - 1) * rz
    assert 0 < last <= rz and last % 8 == 0 and rz % 8 == 0
    mesh = plsc.VectorSubcoreMesh(core_axis_name="c", subcore_axis_name="s")

    @functools.partial(
        pl.kernel,
        out_type=jax.ShapeDtypeStruct((NC, n, d), jnp.float32),
        mesh=mesh,
        scratch_types=[
            pltpu.VMEM((CH,), jnp.int32),
            pltpu.VMEM((CH,), jnp.int32),
            pltpu.VMEM((CH, d), jnp.float32),
            pltpu.VMEM_SHARED((n_acc, d), jnp.float32),
            pltpu.SemaphoreType.DMA,
        ],
    )
    def agg(feat_hbm, src_hbm, dst_hbm, zeros_hbm, out_hbm,
            src_v, dst_v, rows_v, acc_sh, sem):
        c = lax.axis_index("c")
        s = lax.axis_index("s")
        wid = s * NC + c

        z0 = s * rz
        pltpu.sync_copy(zeros_hbm.at[pl.ds(z0, rz)], acc_sh.at[pl.ds(z0, rz)])
        plsc.subcore_barrier()

        base = wid * epw

        def body(k, carry):
            off = base + k * CH
            pltpu.sync_copy(src_hbm.at[pl.ds(off, CH)], src_v)
            pltpu.sync_copy(dst_hbm.at[pl.ds(off, CH)], dst_v)
            pltpu.async_copy(feat_hbm.at[src_v], rows_v, sem).wait()
            pltpu.sync_copy(rows_v, acc_sh.at[dst_v], add=True)
            return carry

        lax.fori_loop(0, n_chunks, body, 0)
        plsc.subcore_barrier()

        r0 = s * rz

        @pl.when(s < NS - 1)
        def _():
            pltpu.sync_copy(acc_sh.at[pl.ds(r0, rz)],
                            out_hbm.at[c, pl.ds(r0, rz)])

        @pl.when(s == NS - 1)
        def _():
            pltpu.sync_copy(acc_sh.at[pl.ds((NS - 1) * rz, last)],
                            out_hbm.at[c, pl.ds((NS - 1) * rz, last)])

    return agg


def _dense_body(x_ref, p0_ref, p1_ref, w_ref, b_ref, o_ref):
    h = x_ref[...] + p0_ref[...] + p1_ref[...]
    y = lax.dot_general(h, w_ref[...], (((1,), (0,)), ((), ())),
                        preferred_element_type=jnp.float32,
                        precision=lax.Precision.HIGHEST)
    o_ref[...] = jnp.maximum(y + b_ref[...], 0.0)


def _make_dense(n, d, h):
    return pl.pallas_call(
        _dense_body,
        out_shape=jax.ShapeDtypeStruct((n, h), jnp.float32),
    )


def _make_final(n, d, h, g, cls):
    def body(h1_ref, p0_ref, p1_ref, w2_ref, b2_ref, gid_ref, wo_ref, bo_ref,
             o_ref):
        x = h1_ref[...] + p0_ref[...] + p1_ref[...]
        y = lax.dot_general(x, w2_ref[...], (((1,), (0,)), ((), ())),
                            preferred_element_type=jnp.float32,
                            precision=lax.Precision.HIGHEST)
        h2 = jnp.maximum(y + b2_ref[...], 0.0)
        gid = gid_ref[...]
        onehot = (gid == lax.broadcasted_iota(jnp.int32, (n, g), 1)
                  ).astype(jnp.float32)
        sums = lax.dot_general(onehot, h2, (((0,), (0,)), ((), ())),
                               preferred_element_type=jnp.float32,
                               precision=lax.Precision.HIGHEST)
        ones = jnp.ones((n, 1), jnp.float32)
        counts = lax.dot_general(onehot, ones, (((0,), (0,)), ((), ())),
                                 preferred_element_type=jnp.float32,
                                 precision=lax.Precision.HIGHEST)
        pooled = sums / jnp.maximum(counts, 1.0)
        logits = lax.dot_general(pooled, wo_ref[...], (((1,), (0,)), ((), ())),
                                 preferred_element_type=jnp.float32,
                                 precision=lax.Precision.HIGHEST) + bo_ref[...]
        m = jnp.max(logits, axis=1, keepdims=True)
        e = jnp.exp(logits - m)
        o_ref[...] = e / jnp.sum(e, axis=1, keepdims=True)

    return pl.pallas_call(
        body,
        out_shape=jax.ShapeDtypeStruct((g, cls), jnp.float32),
    )


def kernel(x, edge_index, i, W1, b1, W2, b2, Wo, bo):
    n, d = x.shape
    hid = W1.shape[1]
    g = 64
    cls = Wo.shape[1]
    e = edge_index.shape[1]

    n_acc = NS * 8 * (-(-(n + 1) // (NS * 8)))
    epw = CH * (-(-e // (NW * CH)))
    e_pad = NW * epw

    src = edge_index[0].astype(jnp.int32)
    dst = edge_index[1].astype(jnp.int32)
    pad = e_pad - e
    if pad:
        src = jnp.concatenate([src, jnp.zeros((pad,), jnp.int32)])
        dst = jnp.concatenate([dst, jnp.full((pad,), n, jnp.int32)])
    zeros = jnp.zeros((n_acc, d), jnp.float32)

    agg = _make_agg(n, n_acc, d, epw)
    dense1 = _make_dense(n, d, hid)
    final = _make_final(n, hid, hid, g, cls)

    p = agg(x, src, dst, zeros)
    h1 = dense1(x, p[0], p[1], W1, b1.reshape(1, -1))
    q = agg(h1, src, dst, zeros)
    return final(h1, q[0], q[1], W2, b2.reshape(1, -1),
                 i.astype(jnp.int32).reshape(-1, 1), Wo, bo.reshape(1, -1))
